# baseline (speedup 1.0000x reference)
import functools

import jax
import jax.numpy as jnp
from jax import lax
from jax.experimental import pallas as pl
from jax.experimental.pallas import tpu as pltpu
from jax.experimental.pallas import tpu_sc as plsc

N = 10000
E = 320000
C1 = 128
C2 = 16
NP = 10240
NC = 2
NS = 16
NW = NC * NS
EPW = E // NW
S = 80
BN = 1024


def _mm_attn_body(x_ref, w_ref, asv_ref, adv_ref, h_ref, sa_ref, da_ref):
    h = jnp.dot(x_ref[...], w_ref[...], preferred_element_type=jnp.float32)
    h_ref[...] = h
    sa_ref[...] = jnp.sum(h * asv_ref[...][None, :], axis=1)
    da_ref[...] = jnp.sum(h * adv_ref[...][None, :], axis=1)


def _mm_attn(x, w, asv, adv, c):
    grid = NP // BN
    return pl.pallas_call(
        _mm_attn_body,
        grid=(grid,),
        in_specs=[
            pl.BlockSpec((BN, C1), lambda i: (i, 0)),
            pl.BlockSpec((C1, c), lambda i: (0, 0)),
            pl.BlockSpec((c,), lambda i: (0,)),
            pl.BlockSpec((c,), lambda i: (0,)),
        ],
        out_specs=[
            pl.BlockSpec((BN, c), lambda i: (i, 0)),
            pl.BlockSpec((BN,), lambda i: (i,)),
            pl.BlockSpec((BN,), lambda i: (i,)),
        ],
        out_shape=[
            jax.ShapeDtypeStruct((NP, c), jnp.float32),
            jax.ShapeDtypeStruct((NP,), jnp.float32),
            jax.ShapeDtypeStruct((NP,), jnp.float32),
        ],
    )(x, w, asv, adv)


def _bridge_body(u_ref, den_ref, b_ref, w_ref, asv_ref, adv_ref,
                 h_ref, sa_ref, da_ref):
    u = u_ref[0] + u_ref[1]
    den = jnp.sum(den_ref[...], axis=0)
    x2 = jnp.maximum(u / (den[:, None] + 1e-16) + b_ref[...][None, :], 0.0)
    h = jnp.dot(x2, w_ref[...], preferred_element_type=jnp.float32)
    h_ref[...] = h
    sa_ref[...] = jnp.sum(h * asv_ref[...][None, :], axis=1)
    da_ref[...] = jnp.sum(h * adv_ref[...][None, :], axis=1)


def _bridge(u, den, b, w, asv, adv):
    grid = NP // BN
    return pl.pallas_call(
        _bridge_body,
        grid=(grid,),
        in_specs=[
            pl.BlockSpec((NC, BN, C1), lambda i: (0, i, 0)),
            pl.BlockSpec((NW, BN), lambda i: (0, i)),
            pl.BlockSpec((C1,), lambda i: (0,)),
            pl.BlockSpec((C1, C2), lambda i: (0, 0)),
            pl.BlockSpec((C2,), lambda i: (0,)),
            pl.BlockSpec((C2,), lambda i: (0,)),
        ],
        out_specs=[
            pl.BlockSpec((BN, C2), lambda i: (i, 0)),
            pl.BlockSpec((BN,), lambda i: (i,)),
            pl.BlockSpec((BN,), lambda i: (i,)),
        ],
        out_shape=[
            jax.ShapeDtypeStruct((NP, C2), jnp.float32),
            jax.ShapeDtypeStruct((NP,), jnp.float32),
            jax.ShapeDtypeStruct((NP,), jnp.float32),
        ],
    )(u, den, b, w, asv, adv)


def _final_body(u_ref, den_ref, b_ref, o_ref):
    u = u_ref[0] + u_ref[1]
    den = jnp.sum(den_ref[...], axis=0)
    z = u / (den[:, None] + 1e-16) + b_ref[...][None, :]
    o_ref[...] = jax.nn.sigmoid(z)


def _final(u, den, b):
    grid = NP // BN
    return pl.pallas_call(
        _final_body,
        grid=(grid,),
        in_specs=[
            pl.BlockSpec((NC, BN, C2), lambda i: (0, i, 0)),
            pl.BlockSpec((NW, BN), lambda i: (0, i)),
            pl.BlockSpec((C2,), lambda i: (0,)),
        ],
        out_specs=pl.BlockSpec((BN, C2), lambda i: (i, 0)),
        out_shape=jax.ShapeDtypeStruct((NP, C2), jnp.float32),
    )(u, den, b)


def _make_logit_kernel():
    mesh = plsc.VectorSubcoreMesh(
        core_axis_name="c", subcore_axis_name="s",
        num_cores=NC, num_subcores=NS)
    G = EPW // S

    @functools.partial(
        pl.kernel,
        mesh=mesh,
        out_type=[
            jax.ShapeDtypeStruct((E,), jnp.float32),
            jax.ShapeDtypeStruct((NW, NP), jnp.float32),
        ],
        scratch_types=[
            pltpu.VMEM((2, S), jnp.int32),
            pltpu.VMEM((2, S), jnp.int32),
            pltpu.VMEM((NP,), jnp.float32),
            pltpu.VMEM((NP,), jnp.float32),
            pltpu.VMEM((NP,), jnp.float32),
            pltpu.VMEM((2, S), jnp.float32),
            pltpu.SemaphoreType.DMA,
            pltpu.SemaphoreType.DMA,
            pltpu.SemaphoreType.DMA,
            pltpu.SemaphoreType.DMA,
            pltpu.SemaphoreType.DMA,
            pltpu.SemaphoreType.DMA,
        ],
        compiler_params=pltpu.CompilerParams(
            needs_layout_passes=False, use_tc_tiling_on_sc=False),
    )
    def logit_kernel(src_e, dst_e, asv, adv, ex_out, den_out,
                     src_v, dst_v, as_v, ad_v, den_v, ex_v,
                     ss0, ss1, sd0, sd1, se0, se1):
        cid = lax.axis_index("c")
        sid = lax.axis_index("s")
        wid = cid * NS + sid
        sem_src = (ss0, ss1)
        sem_dst = (sd0, sd1)
        sem_ex = (se0, se1)
        ebase = wid * EPW

        pltpu.sync_copy(asv, as_v)
        pltpu.sync_copy(adv, ad_v)

        zero16 = jnp.zeros((16,), jnp.float32)

        @pl.loop(0, NP // 16)
        def _(i):
            den_v[pl.ds(i * 16, 16)] = zero16

        def issue_idx(g, b):
            off = ebase + g * S
            pltpu.async_copy(src_e.at[pl.ds(off, S)], src_v.at[b], sem_src[b])
            pltpu.async_copy(dst_e.at[pl.ds(off, S)], dst_v.at[b], sem_dst[b])

        def drain_idx(b):
            pltpu.make_async_copy(
                src_e.at[pl.ds(0, S)], src_v.at[b], sem_src[b]).wait()
            pltpu.make_async_copy(
                dst_e.at[pl.ds(0, S)], dst_v.at[b], sem_dst[b]).wait()

        def drain_ex(b):
            pltpu.make_async_copy(
                ex_v.at[b], ex_out.at[pl.ds(0, S)], sem_ex[b]).wait()

        def scalar_phase(b):
            for i in range(S // 16):
                sl = pl.ds(i * 16, 16)
                si = src_v[b, sl]
                di = dst_v[b, sl]
                av = plsc.load_gather(as_v, [si]) + plsc.load_gather(ad_v, [di])
                av = jnp.where(av >= 0.0, av, 0.2 * av)
                ev = jnp.exp(av)
                ex_v[b, sl] = ev
                plsc.addupdate_scatter(den_v, [di], ev)

        issue_idx(0, 0)
        issue_idx(1, 1)

        @pl.loop(0, G // 2)
        def _(gg):
            for b in range(2):
                g = gg * 2 + b
                drain_idx(b)

                @pl.when(gg > 0)
                def _():
                    drain_ex(b)

                scalar_phase(b)
                pltpu.async_copy(ex_v.at[b],
                                 ex_out.at[pl.ds(ebase + g * S, S)], sem_ex[b])

                @pl.when(g + 2 < G)
                def _():
                    issue_idx(g + 2, b)

        drain_idx(0)
        drain_ex(0)
        scalar_phase(0)
        pltpu.async_copy(ex_v.at[0],
                         ex_out.at[pl.ds(ebase + (G - 1) * S, S)], sem_ex[0])
        drain_ex(1)
        drain_ex(0)

        pltpu.sync_copy(den_v, den_out.at[wid])

    return logit_kernel


def _make_row_kernel(c):
    mesh = plsc.VectorSubcoreMesh(
        core_axis_name="c", subcore_axis_name="s",
        num_cores=NC, num_subcores=NS)
    rpw = NP // NS
    G = EPW // S

    @functools.partial(
        pl.kernel,
        mesh=mesh,
        out_type=jax.ShapeDtypeStruct((NC, NP, c), jnp.float32),
        scratch_types=[
            pltpu.VMEM((2, S), jnp.int32),
            pltpu.VMEM((2, S), jnp.int32),
            pltpu.VMEM((2, S), jnp.int32),
            pltpu.VMEM((2, S), jnp.float32),
            pltpu.VMEM((2, S, c), jnp.float32),
            pltpu.VMEM((2, S, c), jnp.float32),
            pltpu.VMEM_SHARED((NP, c), jnp.float32),
            pltpu.SemaphoreType.DMA,
            pltpu.SemaphoreType.DMA,
            pltpu.SemaphoreType.DMA,
            pltpu.SemaphoreType.DMA,
            pltpu.SemaphoreType.DMA,
            pltpu.SemaphoreType.DMA,
            pltpu.SemaphoreType.DMA,
            pltpu.SemaphoreType.DMA,
            pltpu.SemaphoreType.DMA,
            pltpu.SemaphoreType.DMA,
        ],
        compiler_params=pltpu.CompilerParams(
            needs_layout_passes=False, use_tc_tiling_on_sc=False),
    )
    def row_kernel(src_e, dst_e, ex_e, h, u_out,
                   src_v, dst_v, dsc_v, ex_v, rows_v, sbuf_v, u_sh,
                   ss0, ss1, sd0, sd1, se0, se1, sr0, sr1, sc0, sc1):
        cid = lax.axis_index("c")
        sid = lax.axis_index("s")
        wid = cid * NS + sid
        sem_src = (ss0, ss1)
        sem_dst = (sd0, sd1)
        sem_ex = (se0, se1)
        sem_rows = (sr0, sr1)
        sem_scat = (sc0, sc1)
        ebase = wid * EPW

        zero16 = jnp.zeros((16,), jnp.float32)

        @pl.loop(0, S)
        def _(s):
            for k in range(c // 16):
                rows_v[0, s, pl.ds(k * 16, 16)] = zero16

        @pl.loop(0, rpw // S)
        def _(j):
            pltpu.sync_copy(rows_v.at[0],
                            u_sh.at[pl.ds(sid * rpw + j * S, S)])

        plsc.subcore_barrier()

        def issue_idx(g, b):
            off = ebase + g * S
            pltpu.async_copy(src_e.at[pl.ds(off, S)], src_v.at[b], sem_src[b])
            pltpu.async_copy(dst_e.at[pl.ds(off, S)], dst_v.at[b], sem_dst[b])
            pltpu.async_copy(ex_e.at[pl.ds(off, S)], ex_v.at[b], sem_ex[b])

        def drain_src(b):
            pltpu.make_async_copy(
                src_e.at[pl.ds(0, S)], src_v.at[b], sem_src[b]).wait()

        def drain_dst_ex(b):
            pltpu.make_async_copy(
                dst_e.at[pl.ds(0, S)], dst_v.at[b], sem_dst[b]).wait()
            pltpu.make_async_copy(
                ex_e.at[pl.ds(0, S)], ex_v.at[b], sem_ex[b]).wait()

        def issue_gather(b):
            pltpu.async_copy(h.at[src_v.at[b]], rows_v.at[b], sem_rows[b])

        def drain_gather(b):
            pltpu.make_async_copy(
                h.at[pl.ds(0, S)], rows_v.at[b], sem_rows[b]).wait()

        def issue_scatter(b):
            pltpu.async_copy(sbuf_v.at[b], u_sh.at[dsc_v.at[b]],
                             sem_scat[b], add=True)

        def drain_scatter(b):
            pltpu.make_async_copy(
                h.at[pl.ds(0, S)], sbuf_v.at[b], sem_scat[b]).wait()

        def scale_phase(b):
            @pl.loop(0, S // 16)
            def _(i):
                sl = pl.ds(i * 16, 16)
                dsc_v[b, sl] = dst_v[b, sl]
                ev16 = ex_v[b, sl]
                for j in range(16):
                    evec = lax.broadcast(ev16[j], (16,))
                    for k in range(c // 16):
                        cs = pl.ds(k * 16, 16)
                        sbuf_v[b, i * 16 + j, cs] = (
                            rows_v[b, i * 16 + j, cs] * evec)

        issue_idx(0, 0)
        issue_idx(1, 1)
        drain_src(0)
        issue_gather(0)

        @pl.loop(0, G // 2)
        def _(gg):
            for b in range(2):
                b2 = 1 - b
                g = gg * 2 + b
                drain_src(b2)
                issue_gather(b2)
                drain_gather(b)

                @pl.when(gg > 0)
                def _():
                    drain_scatter(b)

                drain_dst_ex(b)
                scale_phase(b)

                @pl.when(g + 2 < G)
                def _():
                    issue_idx(g + 2, b)

                issue_scatter(b)

        drain_gather(0)
        drain_scatter(0)
        drain_dst_ex(0)
        scale_phase(0)
        issue_scatter(0)
        drain_scatter(1)
        drain_scatter(0)

        plsc.subcore_barrier()

        @pl.loop(0, rpw // S)
        def _(j):
            r0 = sid * rpw + j * S
            pltpu.sync_copy(u_sh.at[pl.ds(r0, S)], u_out.at[cid, pl.ds(r0, S)])

    return row_kernel


def _make_edge_kernel(c):
    mesh = plsc.VectorSubcoreMesh(
        core_axis_name="c", subcore_axis_name="s",
        num_cores=NC, num_subcores=NS)
    rpw = NP // NS
    G = EPW // S

    @functools.partial(
        pl.kernel,
        mesh=mesh,
        out_type=[
            jax.ShapeDtypeStruct((NC, NP, c), jnp.float32),
            jax.ShapeDtypeStruct((NW, NP), jnp.float32),
        ],
        scratch_types=[
            pltpu.VMEM((2, S), jnp.int32),
            pltpu.VMEM((2, S), jnp.int32),
            pltpu.VMEM((2, S), jnp.int32),
            pltpu.VMEM((NP,), jnp.float32),
            pltpu.VMEM((NP,), jnp.float32),
            pltpu.VMEM((NP,), jnp.float32),
            pltpu.VMEM((S,), jnp.float32),
            pltpu.VMEM((2, S, c), jnp.float32),
            pltpu.VMEM((2, S, c), jnp.float32),
            pltpu.VMEM_SHARED((NP, c), jnp.float32),
            pltpu.SemaphoreType.DMA,
            pltpu.SemaphoreType.DMA,
            pltpu.SemaphoreType.DMA,
            pltpu.SemaphoreType.DMA,
            pltpu.SemaphoreType.DMA,
            pltpu.SemaphoreType.DMA,
            pltpu.SemaphoreType.DMA,
            pltpu.SemaphoreType.DMA,
        ],
        compiler_params=pltpu.CompilerParams(
            needs_layout_passes=False, use_tc_tiling_on_sc=False),
    )
    def edge_kernel(src_e, dst_e, asv, adv, h, u_out, den_out,
                    src_v, dst_v, dsc_v, as_v, ad_v, den_v, ex_v,
                    rows_v, sbuf_v, u_sh,
                    ss0, ss1, sd0, sd1, sr0, sr1, sc0, sc1):
        cid = lax.axis_index("c")
        sid = lax.axis_index("s")
        wid = cid * NS + sid
        sem_src = (ss0, ss1)
        sem_dst = (sd0, sd1)
        sem_rows = (sr0, sr1)
        sem_scat = (sc0, sc1)
        ebase = wid * EPW

        pltpu.sync_copy(asv, as_v)
        pltpu.sync_copy(adv, ad_v)

        zero16 = jnp.zeros((16,), jnp.float32)

        @pl.loop(0, NP // 16)
        def _(i):
            den_v[pl.ds(i * 16, 16)] = zero16

        @pl.loop(0, S)
        def _(s):
            for k in range(c // 16):
                rows_v[0, s, pl.ds(k * 16, 16)] = zero16

        @pl.loop(0, rpw // S)
        def _(j):
            pltpu.sync_copy(rows_v.at[0],
                            u_sh.at[pl.ds(sid * rpw + j * S, S)])

        plsc.subcore_barrier()

        def issue_idx(g, b):
            off = ebase + g * S
            pltpu.async_copy(src_e.at[pl.ds(off, S)], src_v.at[b], sem_src[b])
            pltpu.async_copy(dst_e.at[pl.ds(off, S)], dst_v.at[b], sem_dst[b])

        def drain_idx(b):
            pltpu.make_async_copy(
                src_e.at[pl.ds(0, S)], src_v.at[b], sem_src[b]).wait()
            pltpu.make_async_copy(
                dst_e.at[pl.ds(0, S)], dst_v.at[b], sem_dst[b]).wait()

        def issue_gather(b):
            pltpu.async_copy(h.at[src_v.at[b]], rows_v.at[b], sem_rows[b])

        def drain_gather(b):
            pltpu.make_async_copy(
                h.at[pl.ds(0, S)], rows_v.at[b], sem_rows[b]).wait()

        def issue_scatter(b):
            pltpu.async_copy(sbuf_v.at[b], u_sh.at[dsc_v.at[b]],
                             sem_scat[b], add=True)

        def drain_scatter(b):
            pltpu.make_async_copy(
                h.at[pl.ds(0, S)], sbuf_v.at[b], sem_scat[b]).wait()

        def scalar_phase(b):
            for i in range(S // 16):
                sl = pl.ds(i * 16, 16)
                si = src_v[b, sl]
                di = dst_v[b, sl]
                av = plsc.load_gather(as_v, [si]) + plsc.load_gather(ad_v, [di])
                av = jnp.where(av >= 0.0, av, 0.2 * av)
                ev = jnp.exp(av)
                ex_v[sl] = ev
                dsc_v[b, sl] = di
                plsc.addupdate_scatter(den_v, [di], ev)

        def scale_phase(b):
            @pl.loop(0, S // 16)
            def _(i):
                ev16 = ex_v[pl.ds(i * 16, 16)]
                for j in range(16):
                    evec = lax.broadcast(ev16[j], (16,))
                    for k in range(c // 16):
                        cs = pl.ds(k * 16, 16)
                        sbuf_v[b, i * 16 + j, cs] = (
                            rows_v[b, i * 16 + j, cs] * evec)

        issue_idx(0, 0)
        issue_idx(1, 1)
        drain_idx(0)
        issue_gather(0)

        @pl.loop(0, G // 2)
        def _(gg):
            for b in range(2):
                b2 = 1 - b
                g = gg * 2 + b
                drain_idx(b2)
                issue_gather(b2)
                drain_gather(b)

                @pl.when(gg > 0)
                def _():
                    drain_scatter(b)

                scalar_phase(b)

                @pl.when(g + 2 < G)
                def _():
                    issue_idx(g + 2, b)

                scale_phase(b)
                issue_scatter(b)

        drain_gather(0)
        drain_scatter(0)
        scalar_phase(0)
        scale_phase(0)
        issue_scatter(0)
        drain_scatter(1)
        drain_scatter(0)

        pltpu.sync_copy(den_v, den_out.at[wid])
        plsc.subcore_barrier()

        @pl.loop(0, rpw // S)
        def _(j):
            r0 = sid * rpw + j * S
            pltpu.sync_copy(u_sh.at[pl.ds(r0, S)], u_out.at[cid, pl.ds(r0, S)])

    return edge_kernel


_logit1 = _make_logit_kernel()
_rows1 = _make_row_kernel(C1)
_edge2 = _make_edge_kernel(C2)


def kernel(edge_index, embed, W1, a_src1, a_dst1, b1, W2, a_src2, a_dst2, b2):
    ei = edge_index.astype(jnp.int32)
    src_e = ei[0]
    dst_e = ei[1]
    x = jnp.zeros((NP, C1), jnp.float32).at[:N].set(embed)
    h1, sa1, da1 = _mm_attn(x, W1, a_src1, a_dst1, C1)
    ex1, den1 = _logit1(src_e, dst_e, sa1, da1)
    u1 = _rows1(src_e, dst_e, ex1, h1)
    h2, sa2, da2 = _bridge(u1, den1, b1, W2, a_src2, a_dst2)
    u2, den2 = _edge2(src_e, dst_e, sa2, da2, h2)
    out = _final(u2, den2, b2)
    return out[:N]


# speedup vs baseline: 1.1370x; 1.1370x over previous
import functools

import jax
import jax.numpy as jnp
from jax import lax
from jax.experimental import pallas as pl
from jax.experimental.pallas import tpu as pltpu
from jax.experimental.pallas import tpu_sc as plsc

N = 10000
E = 320000
C1 = 128
C2 = 16
NP = 10240
NC = 2
NS = 16
NW = NC * NS
EPW = E // NW
S = 80
BN = 1024


def _mm_attn_body(x_ref, w_ref, asv_ref, adv_ref, h_ref, sa_ref, da_ref):
    h = jnp.dot(x_ref[...], w_ref[...], preferred_element_type=jnp.float32)
    h_ref[...] = h
    sa_ref[...] = jnp.sum(h * asv_ref[...][None, :], axis=1)
    da_ref[...] = jnp.sum(h * adv_ref[...][None, :], axis=1)


def _mm_attn(x, w, asv, adv, c):
    grid = NP // BN
    return pl.pallas_call(
        _mm_attn_body,
        grid=(grid,),
        in_specs=[
            pl.BlockSpec((BN, C1), lambda i: (i, 0)),
            pl.BlockSpec((C1, c), lambda i: (0, 0)),
            pl.BlockSpec((c,), lambda i: (0,)),
            pl.BlockSpec((c,), lambda i: (0,)),
        ],
        out_specs=[
            pl.BlockSpec((BN, c), lambda i: (i, 0)),
            pl.BlockSpec((BN,), lambda i: (i,)),
            pl.BlockSpec((BN,), lambda i: (i,)),
        ],
        out_shape=[
            jax.ShapeDtypeStruct((NP, c), jnp.float32),
            jax.ShapeDtypeStruct((NP,), jnp.float32),
            jax.ShapeDtypeStruct((NP,), jnp.float32),
        ],
    )(x, w, asv, adv)


def _bridge_body(u_ref, den_ref, b_ref, w_ref, asv_ref, adv_ref,
                 h_ref, sa_ref, da_ref):
    u = u_ref[0] + u_ref[1]
    den = jnp.sum(den_ref[...], axis=0)
    x2 = jnp.maximum(u / (den[:, None] + 1e-16) + b_ref[...][None, :], 0.0)
    h = jnp.dot(x2, w_ref[...], preferred_element_type=jnp.float32)
    h_ref[...] = h
    sa_ref[...] = jnp.sum(h * asv_ref[...][None, :], axis=1)
    da_ref[...] = jnp.sum(h * adv_ref[...][None, :], axis=1)


def _bridge(u, den, b, w, asv, adv):
    grid = NP // BN
    return pl.pallas_call(
        _bridge_body,
        grid=(grid,),
        in_specs=[
            pl.BlockSpec((NC, BN, C1), lambda i: (0, i, 0)),
            pl.BlockSpec((NC, BN), lambda i: (0, i)),
            pl.BlockSpec((C1,), lambda i: (0,)),
            pl.BlockSpec((C1, C2), lambda i: (0, 0)),
            pl.BlockSpec((C2,), lambda i: (0,)),
            pl.BlockSpec((C2,), lambda i: (0,)),
        ],
        out_specs=[
            pl.BlockSpec((BN, C2), lambda i: (i, 0)),
            pl.BlockSpec((BN,), lambda i: (i,)),
            pl.BlockSpec((BN,), lambda i: (i,)),
        ],
        out_shape=[
            jax.ShapeDtypeStruct((NP, C2), jnp.float32),
            jax.ShapeDtypeStruct((NP,), jnp.float32),
            jax.ShapeDtypeStruct((NP,), jnp.float32),
        ],
    )(u, den, b, w, asv, adv)


def _final_body(u_ref, den_ref, b_ref, o_ref):
    u = u_ref[0] + u_ref[1]
    den = jnp.sum(den_ref[...], axis=0)
    z = u / (den[:, None] + 1e-16) + b_ref[...][None, :]
    o_ref[...] = jax.nn.sigmoid(z)


def _final(u, den, b):
    grid = NP // BN
    return pl.pallas_call(
        _final_body,
        grid=(grid,),
        in_specs=[
            pl.BlockSpec((NC, BN, C2), lambda i: (0, i, 0)),
            pl.BlockSpec((NC, BN), lambda i: (0, i)),
            pl.BlockSpec((C2,), lambda i: (0,)),
        ],
        out_specs=pl.BlockSpec((BN, C2), lambda i: (i, 0)),
        out_shape=jax.ShapeDtypeStruct((NP, C2), jnp.float32),
    )(u, den, b)


def _make_logit_kernel():
    mesh = plsc.VectorSubcoreMesh(
        core_axis_name="c", subcore_axis_name="s",
        num_cores=NC, num_subcores=NS)
    G = EPW // S

    @functools.partial(
        pl.kernel,
        mesh=mesh,
        out_type=[
            jax.ShapeDtypeStruct((E,), jnp.float32),
            jax.ShapeDtypeStruct((NW, NP), jnp.float32),
        ],
        scratch_types=[
            pltpu.VMEM((2, S), jnp.int32),
            pltpu.VMEM((2, S), jnp.int32),
            pltpu.VMEM((NP,), jnp.float32),
            pltpu.VMEM((NP,), jnp.float32),
            pltpu.VMEM((NP,), jnp.float32),
            pltpu.VMEM((2, S), jnp.float32),
            pltpu.SemaphoreType.DMA,
            pltpu.SemaphoreType.DMA,
            pltpu.SemaphoreType.DMA,
            pltpu.SemaphoreType.DMA,
            pltpu.SemaphoreType.DMA,
            pltpu.SemaphoreType.DMA,
        ],
        compiler_params=pltpu.CompilerParams(
            needs_layout_passes=False, use_tc_tiling_on_sc=False),
    )
    def logit_kernel(src_e, dst_e, asv, adv, ex_out, den_out,
                     src_v, dst_v, as_v, ad_v, den_v, ex_v,
                     ss0, ss1, sd0, sd1, se0, se1):
        cid = lax.axis_index("c")
        sid = lax.axis_index("s")
        wid = cid * NS + sid
        sem_src = (ss0, ss1)
        sem_dst = (sd0, sd1)
        sem_ex = (se0, se1)
        ebase = wid * EPW

        pltpu.sync_copy(asv, as_v)
        pltpu.sync_copy(adv, ad_v)

        zero16 = jnp.zeros((16,), jnp.float32)

        @pl.loop(0, NP // 16)
        def _(i):
            den_v[pl.ds(i * 16, 16)] = zero16

        def issue_idx(g, b):
            off = ebase + g * S
            pltpu.async_copy(src_e.at[pl.ds(off, S)], src_v.at[b], sem_src[b])
            pltpu.async_copy(dst_e.at[pl.ds(off, S)], dst_v.at[b], sem_dst[b])

        def drain_idx(b):
            pltpu.make_async_copy(
                src_e.at[pl.ds(0, S)], src_v.at[b], sem_src[b]).wait()
            pltpu.make_async_copy(
                dst_e.at[pl.ds(0, S)], dst_v.at[b], sem_dst[b]).wait()

        def drain_ex(b):
            pltpu.make_async_copy(
                ex_v.at[b], ex_out.at[pl.ds(0, S)], sem_ex[b]).wait()

        def scalar_phase(b):
            for i in range(S // 16):
                sl = pl.ds(i * 16, 16)
                si = src_v[b, sl]
                di = dst_v[b, sl]
                av = plsc.load_gather(as_v, [si]) + plsc.load_gather(ad_v, [di])
                av = jnp.where(av >= 0.0, av, 0.2 * av)
                ev = jnp.exp(av)
                ex_v[b, sl] = ev
                plsc.addupdate_scatter(den_v, [di], ev)

        issue_idx(0, 0)
        issue_idx(1, 1)

        @pl.loop(0, G // 2)
        def _(gg):
            for b in range(2):
                g = gg * 2 + b
                drain_idx(b)

                @pl.when(gg > 0)
                def _():
                    drain_ex(b)

                scalar_phase(b)
                pltpu.async_copy(ex_v.at[b],
                                 ex_out.at[pl.ds(ebase + g * S, S)], sem_ex[b])

                @pl.when(g + 2 < G)
                def _():
                    issue_idx(g + 2, b)

        drain_idx(0)
        drain_ex(0)
        scalar_phase(0)
        pltpu.async_copy(ex_v.at[0],
                         ex_out.at[pl.ds(ebase + (G - 1) * S, S)], sem_ex[0])
        drain_ex(1)
        drain_ex(0)

        pltpu.sync_copy(den_v, den_out.at[wid])

    return logit_kernel


def _make_row_kernel(c):
    mesh = plsc.VectorSubcoreMesh(
        core_axis_name="c", subcore_axis_name="s",
        num_cores=NC, num_subcores=NS)
    rpw = NP // NS
    G = EPW // S

    @functools.partial(
        pl.kernel,
        mesh=mesh,
        out_type=jax.ShapeDtypeStruct((NC, NP, c), jnp.float32),
        scratch_types=[
            pltpu.VMEM((2, S), jnp.int32),
            pltpu.VMEM((2, S), jnp.int32),
            pltpu.VMEM((2, S), jnp.int32),
            pltpu.VMEM((2, S), jnp.float32),
            pltpu.VMEM((2, S, c), jnp.float32),
            pltpu.VMEM((2, S, c), jnp.float32),
            pltpu.VMEM_SHARED((NP, c), jnp.float32),
            pltpu.SemaphoreType.DMA,
            pltpu.SemaphoreType.DMA,
            pltpu.SemaphoreType.DMA,
            pltpu.SemaphoreType.DMA,
            pltpu.SemaphoreType.DMA,
            pltpu.SemaphoreType.DMA,
            pltpu.SemaphoreType.DMA,
            pltpu.SemaphoreType.DMA,
            pltpu.SemaphoreType.DMA,
            pltpu.SemaphoreType.DMA,
        ],
        compiler_params=pltpu.CompilerParams(
            needs_layout_passes=False, use_tc_tiling_on_sc=False),
    )
    def row_kernel(src_e, dst_e, ex_e, h, u_out,
                   src_v, dst_v, dsc_v, ex_v, rows_v, sbuf_v, u_sh,
                   ss0, ss1, sd0, sd1, se0, se1, sr0, sr1, sc0, sc1):
        cid = lax.axis_index("c")
        sid = lax.axis_index("s")
        wid = cid * NS + sid
        sem_src = (ss0, ss1)
        sem_dst = (sd0, sd1)
        sem_ex = (se0, se1)
        sem_rows = (sr0, sr1)
        sem_scat = (sc0, sc1)
        ebase = wid * EPW

        zero16 = jnp.zeros((16,), jnp.float32)

        @pl.loop(0, S)
        def _(s):
            for k in range(c // 16):
                rows_v[0, s, pl.ds(k * 16, 16)] = zero16

        @pl.loop(0, rpw // S)
        def _(j):
            pltpu.sync_copy(rows_v.at[0],
                            u_sh.at[pl.ds(sid * rpw + j * S, S)])

        plsc.subcore_barrier()

        def issue_idx(g, b):
            off = ebase + g * S
            pltpu.async_copy(src_e.at[pl.ds(off, S)], src_v.at[b], sem_src[b])
            pltpu.async_copy(dst_e.at[pl.ds(off, S)], dst_v.at[b], sem_dst[b])
            pltpu.async_copy(ex_e.at[pl.ds(off, S)], ex_v.at[b], sem_ex[b])

        def drain_src(b):
            pltpu.make_async_copy(
                src_e.at[pl.ds(0, S)], src_v.at[b], sem_src[b]).wait()

        def drain_dst_ex(b):
            pltpu.make_async_copy(
                dst_e.at[pl.ds(0, S)], dst_v.at[b], sem_dst[b]).wait()
            pltpu.make_async_copy(
                ex_e.at[pl.ds(0, S)], ex_v.at[b], sem_ex[b]).wait()

        def issue_gather(b):
            pltpu.async_copy(h.at[src_v.at[b]], rows_v.at[b], sem_rows[b])

        def drain_gather(b):
            pltpu.make_async_copy(
                h.at[pl.ds(0, S)], rows_v.at[b], sem_rows[b]).wait()

        def issue_scatter(b):
            pltpu.async_copy(sbuf_v.at[b], u_sh.at[dsc_v.at[b]],
                             sem_scat[b], add=True)

        def drain_scatter(b):
            pltpu.make_async_copy(
                h.at[pl.ds(0, S)], sbuf_v.at[b], sem_scat[b]).wait()

        def scale_phase(b):
            @pl.loop(0, S // 16)
            def _(i):
                sl = pl.ds(i * 16, 16)
                dsc_v[b, sl] = dst_v[b, sl]
                ev16 = ex_v[b, sl]
                for j in range(16):
                    evec = lax.broadcast(ev16[j], (16,))
                    for k in range(c // 16):
                        cs = pl.ds(k * 16, 16)
                        sbuf_v[b, i * 16 + j, cs] = (
                            rows_v[b, i * 16 + j, cs] * evec)

        issue_idx(0, 0)
        issue_idx(1, 1)
        drain_src(0)
        issue_gather(0)

        @pl.loop(0, G // 2)
        def _(gg):
            for b in range(2):
                b2 = 1 - b
                g = gg * 2 + b
                drain_src(b2)
                issue_gather(b2)
                drain_gather(b)

                @pl.when(gg > 0)
                def _():
                    drain_scatter(b)

                drain_dst_ex(b)
                scale_phase(b)

                @pl.when(g + 2 < G)
                def _():
                    issue_idx(g + 2, b)

                issue_scatter(b)

        drain_gather(0)
        drain_scatter(0)
        drain_dst_ex(0)
        scale_phase(0)
        issue_scatter(0)
        drain_scatter(1)
        drain_scatter(0)

        plsc.subcore_barrier()

        @pl.loop(0, rpw // S)
        def _(j):
            r0 = sid * rpw + j * S
            pltpu.sync_copy(u_sh.at[pl.ds(r0, S)], u_out.at[cid, pl.ds(r0, S)])

    return row_kernel


def _make_fused_kernel(c):
    mesh = plsc.VectorSubcoreMesh(
        core_axis_name="c", subcore_axis_name="s",
        num_cores=NC, num_subcores=NS)
    rpw = NP // NS
    G = EPW // S

    @functools.partial(
        pl.kernel,
        mesh=mesh,
        out_type=[
            jax.ShapeDtypeStruct((NC, NP, c), jnp.float32),
            jax.ShapeDtypeStruct((NC, NP), jnp.float32),
        ],
        scratch_types=[
            pltpu.VMEM((2, S), jnp.int32),
            pltpu.VMEM((2, S), jnp.int32),
            pltpu.VMEM((2, S), jnp.int32),
            pltpu.VMEM((2, S), jnp.float32),
            pltpu.VMEM((2, S), jnp.float32),
            pltpu.VMEM((2, S), jnp.float32),
            pltpu.VMEM((2, S, c), jnp.float32),
            pltpu.VMEM((2, S, c), jnp.float32),
            pltpu.VMEM_SHARED((NP,), jnp.float32),
            pltpu.VMEM_SHARED((NP,), jnp.float32),
            pltpu.VMEM_SHARED((NP,), jnp.float32),
            pltpu.VMEM_SHARED((NP, c), jnp.float32),
            pltpu.SemaphoreType.DMA,
            pltpu.SemaphoreType.DMA,
            pltpu.SemaphoreType.DMA,
            pltpu.SemaphoreType.DMA,
            pltpu.SemaphoreType.DMA,
            pltpu.SemaphoreType.DMA,
            pltpu.SemaphoreType.DMA,
            pltpu.SemaphoreType.DMA,
            pltpu.SemaphoreType.DMA,
            pltpu.SemaphoreType.DMA,
            pltpu.SemaphoreType.DMA,
            pltpu.SemaphoreType.DMA,
        ],
        compiler_params=pltpu.CompilerParams(
            needs_layout_passes=False, use_tc_tiling_on_sc=False),
    )
    def fused_kernel(src_e, dst_e, asv, adv, h, u_out, den_out,
                     src_v, dst_v, dsc_v, sag_v, dag_v, exs_v,
                     rows_v, sbuf_v, sa_sh, da_sh, den_sh, u_sh,
                     ss0, ss1, sd0, sd1, sa0, sa1, sr0, sr1,
                     sn0, sn1, sc0, sc1):
        cid = lax.axis_index("c")
        sid = lax.axis_index("s")
        sem_src = (ss0, ss1)
        sem_dst = (sd0, sd1)
        sem_alpha = (sa0, sa1)
        sem_rows = (sr0, sr1)
        sem_den = (sn0, sn1)
        sem_scat = (sc0, sc1)
        wid = cid * NS + sid
        ebase = wid * EPW
        r0 = sid * rpw

        zero16 = jnp.zeros((16,), jnp.float32)

        pltpu.sync_copy(asv.at[pl.ds(r0, rpw)], sa_sh.at[pl.ds(r0, rpw)])
        pltpu.sync_copy(adv.at[pl.ds(r0, rpw)], da_sh.at[pl.ds(r0, rpw)])

        @pl.loop(0, S)
        def _(s):
            for k in range(c // 16):
                rows_v[0, s, pl.ds(k * 16, 16)] = zero16

        @pl.loop(0, S // 16)
        def _(i):
            exs_v[0, pl.ds(i * 16, 16)] = zero16

        @pl.loop(0, rpw // S)
        def _(j):
            pltpu.sync_copy(rows_v.at[0], u_sh.at[pl.ds(r0 + j * S, S)])
            pltpu.sync_copy(exs_v.at[0], den_sh.at[pl.ds(r0 + j * S, S)])

        plsc.subcore_barrier()

        def issue_idx(g, b):
            off = ebase + g * S
            pltpu.async_copy(src_e.at[pl.ds(off, S)], src_v.at[b], sem_src[b])
            pltpu.async_copy(dst_e.at[pl.ds(off, S)], dst_v.at[b], sem_dst[b])

        def drain_idx(b):
            pltpu.make_async_copy(
                src_e.at[pl.ds(0, S)], src_v.at[b], sem_src[b]).wait()
            pltpu.make_async_copy(
                dst_e.at[pl.ds(0, S)], dst_v.at[b], sem_dst[b]).wait()

        def issue_gathers(b):
            pltpu.async_copy(h.at[src_v.at[b]], rows_v.at[b], sem_rows[b])
            pltpu.async_copy(sa_sh.at[src_v.at[b]], sag_v.at[b], sem_alpha[b])
            pltpu.async_copy(da_sh.at[dst_v.at[b]], dag_v.at[b], sem_alpha[b])

        def drain_alpha(b):
            pltpu.make_async_copy(
                sa_sh.at[pl.ds(0, S)], sag_v.at[b], sem_alpha[b]).wait()
            pltpu.make_async_copy(
                da_sh.at[pl.ds(0, S)], dag_v.at[b], sem_alpha[b]).wait()

        def drain_rows(b):
            pltpu.make_async_copy(
                h.at[pl.ds(0, S)], rows_v.at[b], sem_rows[b]).wait()

        def issue_den(b):
            pltpu.async_copy(exs_v.at[b], den_sh.at[dsc_v.at[b]],
                             sem_den[b], add=True)

        def drain_den(b):
            pltpu.make_async_copy(
                exs_v.at[b], den_sh.at[pl.ds(0, S)], sem_den[b]).wait()

        def issue_scatter(b):
            pltpu.async_copy(sbuf_v.at[b], u_sh.at[dsc_v.at[b]],
                             sem_scat[b], add=True)

        def drain_scatter(b):
            pltpu.make_async_copy(
                h.at[pl.ds(0, S)], sbuf_v.at[b], sem_scat[b]).wait()

        def compute_ex(b):
            @pl.loop(0, S // 16)
            def _(i):
                sl = pl.ds(i * 16, 16)
                av = sag_v[b, sl] + dag_v[b, sl]
                av = jnp.where(av >= 0.0, av, 0.2 * av)
                exs_v[b, sl] = jnp.exp(av)
                dsc_v[b, sl] = dst_v[b, sl]

        def scale_phase(b):
            @pl.loop(0, S // 16)
            def _(i):
                sl = pl.ds(i * 16, 16)
                ev16 = exs_v[b, sl]
                for j in range(16):
                    evec = lax.broadcast(ev16[j], (16,))
                    for k in range(c // 16):
                        cs = pl.ds(k * 16, 16)
                        sbuf_v[b, i * 16 + j, cs] = (
                            rows_v[b, i * 16 + j, cs] * evec)

        issue_idx(0, 0)
        issue_idx(1, 1)
        drain_idx(0)
        issue_gathers(0)

        @pl.loop(0, G // 2)
        def _(gg):
            for b in range(2):
                b2 = 1 - b
                g = gg * 2 + b
                drain_idx(b2)
                issue_gathers(b2)

                @pl.when(gg > 0)
                def _():
                    drain_den(b)
                    drain_scatter(b)

                drain_alpha(b)
                compute_ex(b)
                issue_den(b)
                drain_rows(b)

                @pl.when(g + 2 < G)
                def _():
                    issue_idx(g + 2, b)

                scale_phase(b)
                issue_scatter(b)

        drain_den(0)
        drain_scatter(0)
        drain_alpha(0)
        compute_ex(0)
        issue_den(0)
        drain_rows(0)
        scale_phase(0)
        issue_scatter(0)
        drain_den(1)
        drain_scatter(1)
        drain_den(0)
        drain_scatter(0)

        plsc.subcore_barrier()

        @pl.loop(0, rpw // S)
        def _(j):
            rr = r0 + j * S
            pltpu.sync_copy(u_sh.at[pl.ds(rr, S)], u_out.at[cid, pl.ds(rr, S)])
            pltpu.sync_copy(den_sh.at[pl.ds(rr, S)],
                            den_out.at[cid, pl.ds(rr, S)])

    return fused_kernel


def _make_edge_kernel(c):
    mesh = plsc.VectorSubcoreMesh(
        core_axis_name="c", subcore_axis_name="s",
        num_cores=NC, num_subcores=NS)
    rpw = NP // NS
    G = EPW // S

    @functools.partial(
        pl.kernel,
        mesh=mesh,
        out_type=[
            jax.ShapeDtypeStruct((NC, NP, c), jnp.float32),
            jax.ShapeDtypeStruct((NW, NP), jnp.float32),
        ],
        scratch_types=[
            pltpu.VMEM((2, S), jnp.int32),
            pltpu.VMEM((2, S), jnp.int32),
            pltpu.VMEM((2, S), jnp.int32),
            pltpu.VMEM((NP,), jnp.float32),
            pltpu.VMEM((NP,), jnp.float32),
            pltpu.VMEM((NP,), jnp.float32),
            pltpu.VMEM((S,), jnp.float32),
            pltpu.VMEM((2, S, c), jnp.float32),
            pltpu.VMEM((2, S, c), jnp.float32),
            pltpu.VMEM_SHARED((NP, c), jnp.float32),
            pltpu.SemaphoreType.DMA,
            pltpu.SemaphoreType.DMA,
            pltpu.SemaphoreType.DMA,
            pltpu.SemaphoreType.DMA,
            pltpu.SemaphoreType.DMA,
            pltpu.SemaphoreType.DMA,
            pltpu.SemaphoreType.DMA,
            pltpu.SemaphoreType.DMA,
        ],
        compiler_params=pltpu.CompilerParams(
            needs_layout_passes=False, use_tc_tiling_on_sc=False),
    )
    def edge_kernel(src_e, dst_e, asv, adv, h, u_out, den_out,
                    src_v, dst_v, dsc_v, as_v, ad_v, den_v, ex_v,
                    rows_v, sbuf_v, u_sh,
                    ss0, ss1, sd0, sd1, sr0, sr1, sc0, sc1):
        cid = lax.axis_index("c")
        sid = lax.axis_index("s")
        wid = cid * NS + sid
        sem_src = (ss0, ss1)
        sem_dst = (sd0, sd1)
        sem_rows = (sr0, sr1)
        sem_scat = (sc0, sc1)
        ebase = wid * EPW

        pltpu.sync_copy(asv, as_v)
        pltpu.sync_copy(adv, ad_v)

        zero16 = jnp.zeros((16,), jnp.float32)

        @pl.loop(0, NP // 16)
        def _(i):
            den_v[pl.ds(i * 16, 16)] = zero16

        @pl.loop(0, S)
        def _(s):
            for k in range(c // 16):
                rows_v[0, s, pl.ds(k * 16, 16)] = zero16

        @pl.loop(0, rpw // S)
        def _(j):
            pltpu.sync_copy(rows_v.at[0],
                            u_sh.at[pl.ds(sid * rpw + j * S, S)])

        plsc.subcore_barrier()

        def issue_idx(g, b):
            off = ebase + g * S
            pltpu.async_copy(src_e.at[pl.ds(off, S)], src_v.at[b], sem_src[b])
            pltpu.async_copy(dst_e.at[pl.ds(off, S)], dst_v.at[b], sem_dst[b])

        def drain_idx(b):
            pltpu.make_async_copy(
                src_e.at[pl.ds(0, S)], src_v.at[b], sem_src[b]).wait()
            pltpu.make_async_copy(
                dst_e.at[pl.ds(0, S)], dst_v.at[b], sem_dst[b]).wait()

        def issue_gather(b):
            pltpu.async_copy(h.at[src_v.at[b]], rows_v.at[b], sem_rows[b])

        def drain_gather(b):
            pltpu.make_async_copy(
                h.at[pl.ds(0, S)], rows_v.at[b], sem_rows[b]).wait()

        def issue_scatter(b):
            pltpu.async_copy(sbuf_v.at[b], u_sh.at[dsc_v.at[b]],
                             sem_scat[b], add=True)

        def drain_scatter(b):
            pltpu.make_async_copy(
                h.at[pl.ds(0, S)], sbuf_v.at[b], sem_scat[b]).wait()

        def scalar_phase(b):
            for i in range(S // 16):
                sl = pl.ds(i * 16, 16)
                si = src_v[b, sl]
                di = dst_v[b, sl]
                av = plsc.load_gather(as_v, [si]) + plsc.load_gather(ad_v, [di])
                av = jnp.where(av >= 0.0, av, 0.2 * av)
                ev = jnp.exp(av)
                ex_v[sl] = ev
                dsc_v[b, sl] = di
                plsc.addupdate_scatter(den_v, [di], ev)

        def scale_phase(b):
            @pl.loop(0, S // 16)
            def _(i):
                ev16 = ex_v[pl.ds(i * 16, 16)]
                for j in range(16):
                    evec = lax.broadcast(ev16[j], (16,))
                    for k in range(c // 16):
                        cs = pl.ds(k * 16, 16)
                        sbuf_v[b, i * 16 + j, cs] = (
                            rows_v[b, i * 16 + j, cs] * evec)

        issue_idx(0, 0)
        issue_idx(1, 1)
        drain_idx(0)
        issue_gather(0)

        @pl.loop(0, G // 2)
        def _(gg):
            for b in range(2):
                b2 = 1 - b
                g = gg * 2 + b
                drain_idx(b2)
                issue_gather(b2)
                drain_gather(b)

                @pl.when(gg > 0)
                def _():
                    drain_scatter(b)

                scalar_phase(b)

                @pl.when(g + 2 < G)
                def _():
                    issue_idx(g + 2, b)

                scale_phase(b)
                issue_scatter(b)

        drain_gather(0)
        drain_scatter(0)
        scalar_phase(0)
        scale_phase(0)
        issue_scatter(0)
        drain_scatter(1)
        drain_scatter(0)

        pltpu.sync_copy(den_v, den_out.at[wid])
        plsc.subcore_barrier()

        @pl.loop(0, rpw // S)
        def _(j):
            r0 = sid * rpw + j * S
            pltpu.sync_copy(u_sh.at[pl.ds(r0, S)], u_out.at[cid, pl.ds(r0, S)])

    return edge_kernel


_fused1 = _make_fused_kernel(C1)
_fused2 = _make_fused_kernel(C2)


def kernel(edge_index, embed, W1, a_src1, a_dst1, b1, W2, a_src2, a_dst2, b2):
    ei = edge_index.astype(jnp.int32)
    src_e = ei[0]
    dst_e = ei[1]
    x = jnp.zeros((NP, C1), jnp.float32).at[:N].set(embed)
    h1, sa1, da1 = _mm_attn(x, W1, a_src1, a_dst1, C1)
    u1, den1 = _fused1(src_e, dst_e, sa1, da1, h1)
    h2, sa2, da2 = _bridge(u1, den1, b1, W2, a_src2, a_dst2)
    u2, den2 = _fused2(src_e, dst_e, sa2, da2, h2)
    out = _final(u2, den2, b2)
    return out[:N]


# speedup vs baseline: 1.3729x; 1.2074x over previous
import functools

import jax
import jax.numpy as jnp
from jax import lax
from jax.experimental import pallas as pl
from jax.experimental.pallas import tpu as pltpu
from jax.experimental.pallas import tpu_sc as plsc

N = 10000
E = 320000
C1 = 128
C2 = 16
NP = 10240
NC = 2
NS = 16
NW = NC * NS
EPW = E // NW
S = 80
BN = 1024


def _mm_attn_body(x_ref, w_ref, asv_ref, adv_ref, h_ref, sa_ref, da_ref):
    h = jnp.dot(x_ref[...], w_ref[...], preferred_element_type=jnp.float32)
    h_ref[...] = h
    sa_ref[...] = jnp.sum(h * asv_ref[...][None, :], axis=1)
    da_ref[...] = jnp.sum(h * adv_ref[...][None, :], axis=1)


def _mm_attn(x, w, asv, adv, c):
    grid = NP // BN
    return pl.pallas_call(
        _mm_attn_body,
        grid=(grid,),
        in_specs=[
            pl.BlockSpec((BN, C1), lambda i: (i, 0)),
            pl.BlockSpec((C1, c), lambda i: (0, 0)),
            pl.BlockSpec((c,), lambda i: (0,)),
            pl.BlockSpec((c,), lambda i: (0,)),
        ],
        out_specs=[
            pl.BlockSpec((BN, c), lambda i: (i, 0)),
            pl.BlockSpec((BN,), lambda i: (i,)),
            pl.BlockSpec((BN,), lambda i: (i,)),
        ],
        out_shape=[
            jax.ShapeDtypeStruct((NP, c), jnp.float32),
            jax.ShapeDtypeStruct((NP,), jnp.float32),
            jax.ShapeDtypeStruct((NP,), jnp.float32),
        ],
    )(x, w, asv, adv)


def _bridge_body(u_ref, den_ref, b_ref, w_ref, asv_ref, adv_ref,
                 h_ref, sa_ref, da_ref):
    u = u_ref[0] + u_ref[1]
    den = jnp.sum(den_ref[...], axis=0)
    x2 = jnp.maximum(u / (den[:, None] + 1e-16) + b_ref[...][None, :], 0.0)
    h = jnp.dot(x2, w_ref[...], preferred_element_type=jnp.float32)
    h_ref[...] = h
    sa_ref[...] = jnp.sum(h * asv_ref[...][None, :], axis=1)
    da_ref[...] = jnp.sum(h * adv_ref[...][None, :], axis=1)


def _bridge(u, den, b, w, asv, adv):
    grid = NP // BN
    return pl.pallas_call(
        _bridge_body,
        grid=(grid,),
        in_specs=[
            pl.BlockSpec((NC, BN, C1), lambda i: (0, i, 0)),
            pl.BlockSpec((NC, BN), lambda i: (0, i)),
            pl.BlockSpec((C1,), lambda i: (0,)),
            pl.BlockSpec((C1, C2), lambda i: (0, 0)),
            pl.BlockSpec((C2,), lambda i: (0,)),
            pl.BlockSpec((C2,), lambda i: (0,)),
        ],
        out_specs=[
            pl.BlockSpec((BN, C2), lambda i: (i, 0)),
            pl.BlockSpec((BN,), lambda i: (i,)),
            pl.BlockSpec((BN,), lambda i: (i,)),
        ],
        out_shape=[
            jax.ShapeDtypeStruct((NP, C2), jnp.float32),
            jax.ShapeDtypeStruct((NP,), jnp.float32),
            jax.ShapeDtypeStruct((NP,), jnp.float32),
        ],
    )(u, den, b, w, asv, adv)


def _final_body(u_ref, den_ref, b_ref, o_ref):
    u = u_ref[0] + u_ref[1]
    den = jnp.sum(den_ref[...], axis=0)
    z = u / (den[:, None] + 1e-16) + b_ref[...][None, :]
    o_ref[...] = jax.nn.sigmoid(z)


def _final(u, den, b):
    grid = NP // BN
    return pl.pallas_call(
        _final_body,
        grid=(grid,),
        in_specs=[
            pl.BlockSpec((NC, BN, C2), lambda i: (0, i, 0)),
            pl.BlockSpec((NC, BN), lambda i: (0, i)),
            pl.BlockSpec((C2,), lambda i: (0,)),
        ],
        out_specs=pl.BlockSpec((BN, C2), lambda i: (i, 0)),
        out_shape=jax.ShapeDtypeStruct((NP, C2), jnp.float32),
    )(u, den, b)


def _make_logit_kernel():
    mesh = plsc.VectorSubcoreMesh(
        core_axis_name="c", subcore_axis_name="s",
        num_cores=NC, num_subcores=NS)
    G = EPW // S

    @functools.partial(
        pl.kernel,
        mesh=mesh,
        out_type=[
            jax.ShapeDtypeStruct((E,), jnp.float32),
            jax.ShapeDtypeStruct((NW, NP), jnp.float32),
        ],
        scratch_types=[
            pltpu.VMEM((2, S), jnp.int32),
            pltpu.VMEM((2, S), jnp.int32),
            pltpu.VMEM((NP,), jnp.float32),
            pltpu.VMEM((NP,), jnp.float32),
            pltpu.VMEM((NP,), jnp.float32),
            pltpu.VMEM((2, S), jnp.float32),
            pltpu.SemaphoreType.DMA,
            pltpu.SemaphoreType.DMA,
            pltpu.SemaphoreType.DMA,
            pltpu.SemaphoreType.DMA,
            pltpu.SemaphoreType.DMA,
            pltpu.SemaphoreType.DMA,
        ],
        compiler_params=pltpu.CompilerParams(
            needs_layout_passes=False, use_tc_tiling_on_sc=False),
    )
    def logit_kernel(src_e, dst_e, asv, adv, ex_out, den_out,
                     src_v, dst_v, as_v, ad_v, den_v, ex_v,
                     ss0, ss1, sd0, sd1, se0, se1):
        cid = lax.axis_index("c")
        sid = lax.axis_index("s")
        wid = cid * NS + sid
        sem_src = (ss0, ss1)
        sem_dst = (sd0, sd1)
        sem_ex = (se0, se1)
        ebase = wid * EPW

        pltpu.sync_copy(asv, as_v)
        pltpu.sync_copy(adv, ad_v)

        zero16 = jnp.zeros((16,), jnp.float32)

        @pl.loop(0, NP // 16)
        def _(i):
            den_v[pl.ds(i * 16, 16)] = zero16

        def issue_idx(g, b):
            off = ebase + g * S
            pltpu.async_copy(src_e.at[pl.ds(off, S)], src_v.at[b], sem_src[b])
            pltpu.async_copy(dst_e.at[pl.ds(off, S)], dst_v.at[b], sem_dst[b])

        def drain_idx(b):
            pltpu.make_async_copy(
                src_e.at[pl.ds(0, S)], src_v.at[b], sem_src[b]).wait()
            pltpu.make_async_copy(
                dst_e.at[pl.ds(0, S)], dst_v.at[b], sem_dst[b]).wait()

        def drain_ex(b):
            pltpu.make_async_copy(
                ex_v.at[b], ex_out.at[pl.ds(0, S)], sem_ex[b]).wait()

        def scalar_phase(b):
            for i in range(S // 16):
                sl = pl.ds(i * 16, 16)
                si = src_v[b, sl]
                di = dst_v[b, sl]
                av = plsc.load_gather(as_v, [si]) + plsc.load_gather(ad_v, [di])
                av = jnp.where(av >= 0.0, av, 0.2 * av)
                ev = jnp.exp(av)
                ex_v[b, sl] = ev
                plsc.addupdate_scatter(den_v, [di], ev)

        issue_idx(0, 0)
        issue_idx(1, 1)

        @pl.loop(0, G // 2)
        def _(gg):
            for b in range(2):
                g = gg * 2 + b
                drain_idx(b)

                @pl.when(gg > 0)
                def _():
                    drain_ex(b)

                scalar_phase(b)
                pltpu.async_copy(ex_v.at[b],
                                 ex_out.at[pl.ds(ebase + g * S, S)], sem_ex[b])

                @pl.when(g + 2 < G)
                def _():
                    issue_idx(g + 2, b)

        drain_idx(0)
        drain_ex(0)
        scalar_phase(0)
        pltpu.async_copy(ex_v.at[0],
                         ex_out.at[pl.ds(ebase + (G - 1) * S, S)], sem_ex[0])
        drain_ex(1)
        drain_ex(0)

        pltpu.sync_copy(den_v, den_out.at[wid])

    return logit_kernel


def _make_row_kernel(c):
    mesh = plsc.VectorSubcoreMesh(
        core_axis_name="c", subcore_axis_name="s",
        num_cores=NC, num_subcores=NS)
    rpw = NP // NS
    G = EPW // S

    @functools.partial(
        pl.kernel,
        mesh=mesh,
        out_type=jax.ShapeDtypeStruct((NC, NP, c), jnp.float32),
        scratch_types=[
            pltpu.VMEM((2, S), jnp.int32),
            pltpu.VMEM((2, S), jnp.int32),
            pltpu.VMEM((2, S), jnp.int32),
            pltpu.VMEM((2, S), jnp.float32),
            pltpu.VMEM((2, S, c), jnp.float32),
            pltpu.VMEM((2, S, c), jnp.float32),
            pltpu.VMEM_SHARED((NP, c), jnp.float32),
            pltpu.SemaphoreType.DMA,
            pltpu.SemaphoreType.DMA,
            pltpu.SemaphoreType.DMA,
            pltpu.SemaphoreType.DMA,
            pltpu.SemaphoreType.DMA,
            pltpu.SemaphoreType.DMA,
            pltpu.SemaphoreType.DMA,
            pltpu.SemaphoreType.DMA,
            pltpu.SemaphoreType.DMA,
            pltpu.SemaphoreType.DMA,
        ],
        compiler_params=pltpu.CompilerParams(
            needs_layout_passes=False, use_tc_tiling_on_sc=False),
    )
    def row_kernel(src_e, dst_e, ex_e, h, u_out,
                   src_v, dst_v, dsc_v, ex_v, rows_v, sbuf_v, u_sh,
                   ss0, ss1, sd0, sd1, se0, se1, sr0, sr1, sc0, sc1):
        cid = lax.axis_index("c")
        sid = lax.axis_index("s")
        wid = cid * NS + sid
        sem_src = (ss0, ss1)
        sem_dst = (sd0, sd1)
        sem_ex = (se0, se1)
        sem_rows = (sr0, sr1)
        sem_scat = (sc0, sc1)
        ebase = wid * EPW

        zero16 = jnp.zeros((16,), jnp.float32)

        @pl.loop(0, S)
        def _(s):
            for k in range(c // 16):
                rows_v[0, s, pl.ds(k * 16, 16)] = zero16

        @pl.loop(0, rpw // S)
        def _(j):
            pltpu.sync_copy(rows_v.at[0],
                            u_sh.at[pl.ds(sid * rpw + j * S, S)])

        plsc.subcore_barrier()

        def issue_idx(g, b):
            off = ebase + g * S
            pltpu.async_copy(src_e.at[pl.ds(off, S)], src_v.at[b], sem_src[b])
            pltpu.async_copy(dst_e.at[pl.ds(off, S)], dst_v.at[b], sem_dst[b])
            pltpu.async_copy(ex_e.at[pl.ds(off, S)], ex_v.at[b], sem_ex[b])

        def drain_src(b):
            pltpu.make_async_copy(
                src_e.at[pl.ds(0, S)], src_v.at[b], sem_src[b]).wait()

        def drain_dst_ex(b):
            pltpu.make_async_copy(
                dst_e.at[pl.ds(0, S)], dst_v.at[b], sem_dst[b]).wait()
            pltpu.make_async_copy(
                ex_e.at[pl.ds(0, S)], ex_v.at[b], sem_ex[b]).wait()

        def issue_gather(b):
            pltpu.async_copy(h.at[src_v.at[b]], rows_v.at[b], sem_rows[b])

        def drain_gather(b):
            pltpu.make_async_copy(
                h.at[pl.ds(0, S)], rows_v.at[b], sem_rows[b]).wait()

        def issue_scatter(b):
            pltpu.async_copy(sbuf_v.at[b], u_sh.at[dsc_v.at[b]],
                             sem_scat[b], add=True)

        def drain_scatter(b):
            pltpu.make_async_copy(
                h.at[pl.ds(0, S)], sbuf_v.at[b], sem_scat[b]).wait()

        def scale_phase(b):
            @pl.loop(0, S // 16)
            def _(i):
                sl = pl.ds(i * 16, 16)
                dsc_v[b, sl] = dst_v[b, sl]
                ev16 = ex_v[b, sl]
                for j in range(16):
                    evec = lax.broadcast(ev16[j], (16,))
                    for k in range(c // 16):
                        cs = pl.ds(k * 16, 16)
                        sbuf_v[b, i * 16 + j, cs] = (
                            rows_v[b, i * 16 + j, cs] * evec)

        issue_idx(0, 0)
        issue_idx(1, 1)
        drain_src(0)
        issue_gather(0)

        @pl.loop(0, G // 2)
        def _(gg):
            for b in range(2):
                b2 = 1 - b
                g = gg * 2 + b
                drain_src(b2)
                issue_gather(b2)
                drain_gather(b)

                @pl.when(gg > 0)
                def _():
                    drain_scatter(b)

                drain_dst_ex(b)
                scale_phase(b)

                @pl.when(g + 2 < G)
                def _():
                    issue_idx(g + 2, b)

                issue_scatter(b)

        drain_gather(0)
        drain_scatter(0)
        drain_dst_ex(0)
        scale_phase(0)
        issue_scatter(0)
        drain_scatter(1)
        drain_scatter(0)

        plsc.subcore_barrier()

        @pl.loop(0, rpw // S)
        def _(j):
            r0 = sid * rpw + j * S
            pltpu.sync_copy(u_sh.at[pl.ds(r0, S)], u_out.at[cid, pl.ds(r0, S)])

    return row_kernel


def _make_fused_kernel(c, s=S, rows_from_spmem=False):
    mesh = plsc.VectorSubcoreMesh(
        core_axis_name="c", subcore_axis_name="s",
        num_cores=NC, num_subcores=NS)
    rpw = NP // NS
    G = EPW // s

    z = max(d for d in range(16, s + 1, 16) if rpw % d == 0)
    HT = NP if rows_from_spmem else 16

    @functools.partial(
        pl.kernel,
        mesh=mesh,
        out_type=[
            jax.ShapeDtypeStruct((NC, NP, c), jnp.float32),
            jax.ShapeDtypeStruct((NC, NP), jnp.float32),
        ],
        scratch_types=[
            pltpu.VMEM((2, s), jnp.int32),
            pltpu.VMEM((2, s), jnp.int32),
            pltpu.VMEM((2, s), jnp.int32),
            pltpu.VMEM((2, s), jnp.float32),
            pltpu.VMEM((2, s), jnp.float32),
            pltpu.VMEM((2, s), jnp.float32),
            pltpu.VMEM((2, s, c), jnp.float32),
            pltpu.VMEM((2, s, c), jnp.float32),
            pltpu.VMEM_SHARED((NP,), jnp.float32),
            pltpu.VMEM_SHARED((NP,), jnp.float32),
            pltpu.VMEM_SHARED((NP,), jnp.float32),
            pltpu.VMEM_SHARED((NP, c), jnp.float32),
            pltpu.VMEM_SHARED((HT, c), jnp.float32),
            pltpu.SemaphoreType.DMA,
            pltpu.SemaphoreType.DMA,
            pltpu.SemaphoreType.DMA,
            pltpu.SemaphoreType.DMA,
            pltpu.SemaphoreType.DMA,
            pltpu.SemaphoreType.DMA,
            pltpu.SemaphoreType.DMA,
            pltpu.SemaphoreType.DMA,
            pltpu.SemaphoreType.DMA,
            pltpu.SemaphoreType.DMA,
            pltpu.SemaphoreType.DMA,
            pltpu.SemaphoreType.DMA,
        ],
        compiler_params=pltpu.CompilerParams(
            needs_layout_passes=False, use_tc_tiling_on_sc=False),
    )
    def fused_kernel(src_e, dst_e, asv, adv, h, u_out, den_out,
                     src_v, dst_v, dsc_v, sag_v, dag_v, exs_v,
                     rows_v, sbuf_v, sa_sh, da_sh, den_sh, u_sh, h_sh,
                     ss0, ss1, sd0, sd1, sa0, sa1, sr0, sr1,
                     sn0, sn1, sc0, sc1):
        cid = lax.axis_index("c")
        sid = lax.axis_index("s")
        sem_src = (ss0, ss1)
        sem_dst = (sd0, sd1)
        sem_alpha = (sa0, sa1)
        sem_rows = (sr0, sr1)
        sem_den = (sn0, sn1)
        sem_scat = (sc0, sc1)
        wid = cid * NS + sid
        ebase = wid * EPW
        r0 = sid * rpw

        zero16 = jnp.zeros((16,), jnp.float32)

        pltpu.sync_copy(asv.at[pl.ds(r0, rpw)], sa_sh.at[pl.ds(r0, rpw)])
        pltpu.sync_copy(adv.at[pl.ds(r0, rpw)], da_sh.at[pl.ds(r0, rpw)])
        if rows_from_spmem:
            pltpu.sync_copy(h.at[pl.ds(r0, rpw)], h_sh.at[pl.ds(r0, rpw)])

        @pl.loop(0, z)
        def _(t):
            for k in range(c // 16):
                rows_v[0, t, pl.ds(k * 16, 16)] = zero16

        @pl.loop(0, z // 16)
        def _(i):
            exs_v[0, pl.ds(i * 16, 16)] = zero16

        @pl.loop(0, rpw // z)
        def _(j):
            pltpu.sync_copy(rows_v.at[0, pl.ds(0, z)],
                            u_sh.at[pl.ds(r0 + j * z, z)])
            pltpu.sync_copy(exs_v.at[0, pl.ds(0, z)],
                            den_sh.at[pl.ds(r0 + j * z, z)])

        plsc.subcore_barrier()

        def issue_idx(g, b):
            off = ebase + g * s
            pltpu.async_copy(src_e.at[pl.ds(off, s)], src_v.at[b], sem_src[b])
            pltpu.async_copy(dst_e.at[pl.ds(off, s)], dst_v.at[b], sem_dst[b])

        def drain_idx(b):
            pltpu.make_async_copy(
                src_e.at[pl.ds(0, s)], src_v.at[b], sem_src[b]).wait()
            pltpu.make_async_copy(
                dst_e.at[pl.ds(0, s)], dst_v.at[b], sem_dst[b]).wait()

        def issue_gathers(b):
            if rows_from_spmem:
                pltpu.async_copy(h_sh.at[src_v.at[b]], rows_v.at[b],
                                 sem_rows[b])
            else:
                pltpu.async_copy(h.at[src_v.at[b]], rows_v.at[b], sem_rows[b])
            pltpu.async_copy(sa_sh.at[src_v.at[b]], sag_v.at[b], sem_alpha[b])
            pltpu.async_copy(da_sh.at[dst_v.at[b]], dag_v.at[b], sem_alpha[b])

        def drain_alpha(b):
            pltpu.make_async_copy(
                sa_sh.at[pl.ds(0, s)], sag_v.at[b], sem_alpha[b]).wait()
            pltpu.make_async_copy(
                da_sh.at[pl.ds(0, s)], dag_v.at[b], sem_alpha[b]).wait()

        def drain_rows(b):
            pltpu.make_async_copy(
                h.at[pl.ds(0, s)], rows_v.at[b], sem_rows[b]).wait()

        def issue_den(b):
            pltpu.async_copy(exs_v.at[b], den_sh.at[dsc_v.at[b]],
                             sem_den[b], add=True)

        def drain_den(b):
            pltpu.make_async_copy(
                exs_v.at[b], den_sh.at[pl.ds(0, s)], sem_den[b]).wait()

        def issue_scatter(b):
            pltpu.async_copy(sbuf_v.at[b], u_sh.at[dsc_v.at[b]],
                             sem_scat[b], add=True)

        def drain_scatter(b):
            pltpu.make_async_copy(
                h.at[pl.ds(0, s)], sbuf_v.at[b], sem_scat[b]).wait()

        def compute_ex(b):
            @pl.loop(0, s // 16)
            def _(i):
                sl = pl.ds(i * 16, 16)
                av = sag_v[b, sl] + dag_v[b, sl]
                av = jnp.where(av >= 0.0, av, 0.2 * av)
                exs_v[b, sl] = jnp.exp(av)
                dsc_v[b, sl] = dst_v[b, sl]

        def scale_phase(b):
            @pl.loop(0, s // 16)
            def _(i):
                sl = pl.ds(i * 16, 16)
                ev16 = exs_v[b, sl]
                for j in range(16):
                    evec = lax.broadcast(ev16[j], (16,))
                    for k in range(c // 16):
                        cs = pl.ds(k * 16, 16)
                        sbuf_v[b, i * 16 + j, cs] = (
                            rows_v[b, i * 16 + j, cs] * evec)

        issue_idx(0, 0)
        issue_idx(1, 1)
        drain_idx(0)
        issue_gathers(0)

        @pl.loop(0, G // 2)
        def _(gg):
            for b in range(2):
                b2 = 1 - b
                g = gg * 2 + b
                drain_idx(b2)
                issue_gathers(b2)

                @pl.when(gg > 0)
                def _():
                    drain_den(b)
                    drain_scatter(b)

                drain_alpha(b)
                compute_ex(b)
                issue_den(b)
                drain_rows(b)

                @pl.when(g + 2 < G)
                def _():
                    issue_idx(g + 2, b)

                scale_phase(b)
                issue_scatter(b)

        drain_den(0)
        drain_scatter(0)
        drain_alpha(0)
        compute_ex(0)
        issue_den(0)
        drain_rows(0)
        scale_phase(0)
        issue_scatter(0)
        drain_den(1)
        drain_scatter(1)
        drain_den(0)
        drain_scatter(0)

        plsc.subcore_barrier()

        @pl.loop(0, rpw // z)
        def _(j):
            rr = r0 + j * z
            pltpu.sync_copy(u_sh.at[pl.ds(rr, z)], u_out.at[cid, pl.ds(rr, z)])
            pltpu.sync_copy(den_sh.at[pl.ds(rr, z)],
                            den_out.at[cid, pl.ds(rr, z)])

    return fused_kernel


def _make_edge_kernel(c):
    mesh = plsc.VectorSubcoreMesh(
        core_axis_name="c", subcore_axis_name="s",
        num_cores=NC, num_subcores=NS)
    rpw = NP // NS
    G = EPW // S

    @functools.partial(
        pl.kernel,
        mesh=mesh,
        out_type=[
            jax.ShapeDtypeStruct((NC, NP, c), jnp.float32),
            jax.ShapeDtypeStruct((NW, NP), jnp.float32),
        ],
        scratch_types=[
            pltpu.VMEM((2, S), jnp.int32),
            pltpu.VMEM((2, S), jnp.int32),
            pltpu.VMEM((2, S), jnp.int32),
            pltpu.VMEM((NP,), jnp.float32),
            pltpu.VMEM((NP,), jnp.float32),
            pltpu.VMEM((NP,), jnp.float32),
            pltpu.VMEM((S,), jnp.float32),
            pltpu.VMEM((2, S, c), jnp.float32),
            pltpu.VMEM((2, S, c), jnp.float32),
            pltpu.VMEM_SHARED((NP, c), jnp.float32),
            pltpu.SemaphoreType.DMA,
            pltpu.SemaphoreType.DMA,
            pltpu.SemaphoreType.DMA,
            pltpu.SemaphoreType.DMA,
            pltpu.SemaphoreType.DMA,
            pltpu.SemaphoreType.DMA,
            pltpu.SemaphoreType.DMA,
            pltpu.SemaphoreType.DMA,
        ],
        compiler_params=pltpu.CompilerParams(
            needs_layout_passes=False, use_tc_tiling_on_sc=False),
    )
    def edge_kernel(src_e, dst_e, asv, adv, h, u_out, den_out,
                    src_v, dst_v, dsc_v, as_v, ad_v, den_v, ex_v,
                    rows_v, sbuf_v, u_sh,
                    ss0, ss1, sd0, sd1, sr0, sr1, sc0, sc1):
        cid = lax.axis_index("c")
        sid = lax.axis_index("s")
        wid = cid * NS + sid
        sem_src = (ss0, ss1)
        sem_dst = (sd0, sd1)
        sem_rows = (sr0, sr1)
        sem_scat = (sc0, sc1)
        ebase = wid * EPW

        pltpu.sync_copy(asv, as_v)
        pltpu.sync_copy(adv, ad_v)

        zero16 = jnp.zeros((16,), jnp.float32)

        @pl.loop(0, NP // 16)
        def _(i):
            den_v[pl.ds(i * 16, 16)] = zero16

        @pl.loop(0, S)
        def _(s):
            for k in range(c // 16):
                rows_v[0, s, pl.ds(k * 16, 16)] = zero16

        @pl.loop(0, rpw // S)
        def _(j):
            pltpu.sync_copy(rows_v.at[0],
                            u_sh.at[pl.ds(sid * rpw + j * S, S)])

        plsc.subcore_barrier()

        def issue_idx(g, b):
            off = ebase + g * S
            pltpu.async_copy(src_e.at[pl.ds(off, S)], src_v.at[b], sem_src[b])
            pltpu.async_copy(dst_e.at[pl.ds(off, S)], dst_v.at[b], sem_dst[b])

        def drain_idx(b):
            pltpu.make_async_copy(
                src_e.at[pl.ds(0, S)], src_v.at[b], sem_src[b]).wait()
            pltpu.make_async_copy(
                dst_e.at[pl.ds(0, S)], dst_v.at[b], sem_dst[b]).wait()

        def issue_gather(b):
            pltpu.async_copy(h.at[src_v.at[b]], rows_v.at[b], sem_rows[b])

        def drain_gather(b):
            pltpu.make_async_copy(
                h.at[pl.ds(0, S)], rows_v.at[b], sem_rows[b]).wait()

        def issue_scatter(b):
            pltpu.async_copy(sbuf_v.at[b], u_sh.at[dsc_v.at[b]],
                             sem_scat[b], add=True)

        def drain_scatter(b):
            pltpu.make_async_copy(
                h.at[pl.ds(0, S)], sbuf_v.at[b], sem_scat[b]).wait()

        def scalar_phase(b):
            for i in range(S // 16):
                sl = pl.ds(i * 16, 16)
                si = src_v[b, sl]
                di = dst_v[b, sl]
                av = plsc.load_gather(as_v, [si]) + plsc.load_gather(ad_v, [di])
                av = jnp.where(av >= 0.0, av, 0.2 * av)
                ev = jnp.exp(av)
                ex_v[sl] = ev
                dsc_v[b, sl] = di
                plsc.addupdate_scatter(den_v, [di], ev)

        def scale_phase(b):
            @pl.loop(0, S // 16)
            def _(i):
                ev16 = ex_v[pl.ds(i * 16, 16)]
                for j in range(16):
                    evec = lax.broadcast(ev16[j], (16,))
                    for k in range(c // 16):
                        cs = pl.ds(k * 16, 16)
                        sbuf_v[b, i * 16 + j, cs] = (
                            rows_v[b, i * 16 + j, cs] * evec)

        issue_idx(0, 0)
        issue_idx(1, 1)
        drain_idx(0)
        issue_gather(0)

        @pl.loop(0, G // 2)
        def _(gg):
            for b in range(2):
                b2 = 1 - b
                g = gg * 2 + b
                drain_idx(b2)
                issue_gather(b2)
                drain_gather(b)

                @pl.when(gg > 0)
                def _():
                    drain_scatter(b)

                scalar_phase(b)

                @pl.when(g + 2 < G)
                def _():
                    issue_idx(g + 2, b)

                scale_phase(b)
                issue_scatter(b)

        drain_gather(0)
        drain_scatter(0)
        scalar_phase(0)
        scale_phase(0)
        issue_scatter(0)
        drain_scatter(1)
        drain_scatter(0)

        pltpu.sync_copy(den_v, den_out.at[wid])
        plsc.subcore_barrier()

        @pl.loop(0, rpw // S)
        def _(j):
            r0 = sid * rpw + j * S
            pltpu.sync_copy(u_sh.at[pl.ds(r0, S)], u_out.at[cid, pl.ds(r0, S)])

    return edge_kernel


_fused1 = _make_fused_kernel(C1)
_fused2 = _make_fused_kernel(C2, s=400, rows_from_spmem=True)


def kernel(edge_index, embed, W1, a_src1, a_dst1, b1, W2, a_src2, a_dst2, b2):
    ei = edge_index.astype(jnp.int32)
    src_e = ei[0]
    dst_e = ei[1]
    x = jnp.zeros((NP, C1), jnp.float32).at[:N].set(embed)
    h1, sa1, da1 = _mm_attn(x, W1, a_src1, a_dst1, C1)
    u1, den1 = _fused1(src_e, dst_e, sa1, da1, h1)
    h2, sa2, da2 = _bridge(u1, den1, b1, W2, a_src2, a_dst2)
    u2, den2 = _fused2(src_e, dst_e, sa2, da2, h2)
    out = _final(u2, den2, b2)
    return out[:N]
